# baseline (speedup 1.0000x reference)
import functools

import jax
import jax.numpy as jnp
from jax.experimental import pallas as pl

_ROWS = 16384
_D = 32
_K = 8192
_R = 512
_T = _ROWS // _R


def _vq_tile(flat_ref, embt_ref, ecat_ref, fsq_ref, esq_ref,
             zq_ref, idx_ref, loss_ref):
    i = pl.program_id(0)
    flat = flat_ref[...]
    mm = jax.lax.dot_general(
        flat.astype(jnp.bfloat16), embt_ref[...].astype(jnp.bfloat16),
        (((1,), (0,)), ((), ())),
        preferred_element_type=jnp.float32)
    dist = (fsq_ref[...] - 2.0 * mm) + esq_ref[...]
    blk = _K // 2
    iota = jax.lax.broadcasted_iota(jnp.int32, (_R, _K), 1)
    acc = jnp.full((_R,), jnp.inf, jnp.float32)
    idx = jnp.zeros((_R,), jnp.int32)
    for b in range(2):
        db = dist[:, b * blk:(b + 1) * blk]
        m = jnp.min(db, axis=1)
        jm = jnp.min(jnp.where(db == m[:, None],
                               iota[:, b * blk:(b + 1) * blk], _K), axis=1)
        take = m < acc
        idx = jnp.where(take, jm, idx)
        acc = jnp.where(take, m.astype(jnp.bfloat16).astype(jnp.float32), acc)
    onehot = (iota == idx[:, None]).astype(jnp.bfloat16)
    zqcat = jax.lax.dot_general(
        onehot, ecat_ref[...], (((1,), (0,)), ((), ())),
        preferred_element_type=jnp.float32)
    zq = zqcat[:, :_D] + zqcat[:, _D:]
    zq_ref[...] = flat + (zq - flat)
    idx_ref[0, 0, :] = idx
    part = jnp.sum((flat - zq) ** 2).reshape(1, 1)

    @pl.when(i == 0)
    def _init():
        loss_ref[...] = part

    @pl.when(i != 0)
    def _acc():
        loss_ref[...] += part


@functools.partial(jax.jit, static_argnames=("interpret",))
def kernel(z_e, embedding, interpret=False):
    B, T, D = z_e.shape
    flat = z_e.reshape(-1, D)
    fsq = jnp.sum(flat ** 2, axis=1, keepdims=True)
    esq = jnp.sum(embedding ** 2, axis=1)[None, :]
    embt = embedding.T
    ehi = embedding.astype(jnp.bfloat16)
    elo = (embedding - ehi.astype(jnp.float32)).astype(jnp.bfloat16)
    ecat = jnp.concatenate([ehi, elo], axis=1)

    zq, idx, loss_sum = pl.pallas_call(
        _vq_tile,
        grid=(_T,),
        in_specs=[
            pl.BlockSpec((_R, _D), lambda i: (i, 0)),
            pl.BlockSpec((_D, _K), lambda i: (0, 0)),
            pl.BlockSpec((_K, 2 * _D), lambda i: (0, 0)),
            pl.BlockSpec((_R, 1), lambda i: (i, 0)),
            pl.BlockSpec((1, _K), lambda i: (0, 0)),
        ],
        out_specs=[
            pl.BlockSpec((_R, _D), lambda i: (i, 0)),
            pl.BlockSpec((1, 1, _R), lambda i: (i, 0, 0)),
            pl.BlockSpec((1, 1), lambda i: (0, 0)),
        ],
        out_shape=[
            jax.ShapeDtypeStruct((_ROWS, _D), jnp.float32),
            jax.ShapeDtypeStruct((_T, 1, _R), jnp.int32),
            jax.ShapeDtypeStruct((1, 1), jnp.float32),
        ],
        interpret=interpret,
    )(flat, embt, ecat, fsq, esq)

    z_q_st = zq.reshape(B, T, D)
    indices = idx.reshape(B, T)
    loss = loss_sum[0, 0] * (1.0 / (_ROWS * _D))
    return (z_q_st, indices, loss, loss)


# speedup vs baseline: 1.3406x; 1.3406x over previous
import functools

import jax
import jax.numpy as jnp
from jax.experimental import pallas as pl

_ROWS = 16384
_D = 32
_K = 8192
_R = 256
_T = _ROWS // _R


def _vq_tile(flat_ref, embt_ref, ecat_ref, fsq_ref, esq_ref,
             zq_ref, idx_ref, loss_ref):
    i = pl.program_id(0)
    flat = flat_ref[...]
    mm = jax.lax.dot_general(
        flat.astype(jnp.bfloat16), embt_ref[...].astype(jnp.bfloat16),
        (((1,), (0,)), ((), ())),
        preferred_element_type=jnp.float32)
    dist = (fsq_ref[...] - 2.0 * mm) + esq_ref[...]
    blk = _K // 2
    iota = jax.lax.broadcasted_iota(jnp.int32, (_R, _K), 1)
    acc = jnp.full((_R,), jnp.inf, jnp.float32)
    idx = jnp.zeros((_R,), jnp.int32)
    for b in range(2):
        db = dist[:, b * blk:(b + 1) * blk]
        m = jnp.min(db, axis=1)
        jm = jnp.min(jnp.where(db == m[:, None],
                               iota[:, b * blk:(b + 1) * blk], _K), axis=1)
        take = m < acc
        idx = jnp.where(take, jm, idx)
        acc = jnp.where(take, m.astype(jnp.bfloat16).astype(jnp.float32), acc)
    onehot = (iota == idx[:, None]).astype(jnp.bfloat16)
    zqcat = jax.lax.dot_general(
        onehot, ecat_ref[...], (((1,), (0,)), ((), ())),
        preferred_element_type=jnp.float32)
    zq = zqcat[:, :_D] + zqcat[:, _D:]
    zq_ref[...] = flat + (zq - flat)
    idx_ref[0, 0, :] = idx
    part = jnp.sum((flat - zq) ** 2).reshape(1, 1)

    @pl.when(i == 0)
    def _init():
        loss_ref[...] = part

    @pl.when(i != 0)
    def _acc():
        loss_ref[...] += part


@functools.partial(jax.jit, static_argnames=("interpret",))
def kernel(z_e, embedding, interpret=False):
    B, T, D = z_e.shape
    flat = z_e.reshape(-1, D)
    fsq = jnp.sum(flat ** 2, axis=1, keepdims=True)
    esq = jnp.sum(embedding ** 2, axis=1)[None, :]
    embt = embedding.T
    ehi = embedding.astype(jnp.bfloat16)
    elo = (embedding - ehi.astype(jnp.float32)).astype(jnp.bfloat16)
    ecat = jnp.concatenate([ehi, elo], axis=1)

    zq, idx, loss_sum = pl.pallas_call(
        _vq_tile,
        grid=(_T,),
        in_specs=[
            pl.BlockSpec((_R, _D), lambda i: (i, 0)),
            pl.BlockSpec((_D, _K), lambda i: (0, 0)),
            pl.BlockSpec((_K, 2 * _D), lambda i: (0, 0)),
            pl.BlockSpec((_R, 1), lambda i: (i, 0)),
            pl.BlockSpec((1, _K), lambda i: (0, 0)),
        ],
        out_specs=[
            pl.BlockSpec((_R, _D), lambda i: (i, 0)),
            pl.BlockSpec((1, 1, _R), lambda i: (i, 0, 0)),
            pl.BlockSpec((1, 1), lambda i: (0, 0)),
        ],
        out_shape=[
            jax.ShapeDtypeStruct((_ROWS, _D), jnp.float32),
            jax.ShapeDtypeStruct((_T, 1, _R), jnp.int32),
            jax.ShapeDtypeStruct((1, 1), jnp.float32),
        ],
        interpret=interpret,
    )(flat, embt, ecat, fsq, esq)

    z_q_st = zq.reshape(B, T, D)
    indices = idx.reshape(B, T)
    loss = loss_sum[0, 0] * (1.0 / (_ROWS * _D))
    return (z_q_st, indices, loss, loss)
